# baseline (speedup 1.0000x reference)
import jax
import jax.numpy as jnp
from jax import lax
from jax.experimental import pallas as pl
from jax.experimental.pallas import tpu as pltpu
from jax.experimental.pallas import tpu_sc as plsc

PAIRS = [[1,0],[0,5],[1,3],[0,2],[1,7],[0,11],[1,1],[0,0],[1,9],[0,7],[1,4],[0,9],[1,12],[0,3],[1,6],[0,14],[1,2],[0,1],[1,15],[0,13],[1,8],[0,6],[1,10],[0,4],[1,5],[0,8],[1,14],[0,10],[1,13],[0,12],[1,11],[0,15]]

_A = [o for l, o in PAIRS if l == 1]
_B = [o for l, o in PAIRS if l == 0]

_NUM_CORES = 2
_NUM_SUBCORES = 16
_D = 256


def _body(l1_hbm, l0_hbm, out_hbm, row_v, blk_v, sem):
    w = lax.axis_index("s") * _NUM_CORES + lax.axis_index("c")

    for i in range(16):

        @pl.when(w == i)
        def _(i=i):
            pltpu.sync_copy(l1_hbm.at[pl.ds(_A[i], 1)], row_v)
            cps = [
                pltpu.async_copy(row_v, out_hbm.at[pl.ds(16 * i + j, 1)], sem)
                for j in range(8)
            ]
            for c in cps:
                c.wait()

    for i in range(16):

        @pl.when(w == 16 + i)
        def _(i=i):
            pltpu.sync_copy(l0_hbm.at[pl.ds(8 * _B[i], 8)], blk_v)
            pltpu.sync_copy(blk_v, out_hbm.at[pl.ds(16 * i + 8, 8)])


def _make_sc_gather():
    return pl.kernel(
        _body,
        out_type=jax.ShapeDtypeStruct((256, _D), jnp.float32),
        mesh=plsc.VectorSubcoreMesh(
            core_axis_name="c",
            subcore_axis_name="s",
            num_cores=_NUM_CORES,
            num_subcores=_NUM_SUBCORES,
        ),
        scratch_types=[
            pltpu.VMEM((1, _D), jnp.float32),
            pltpu.VMEM((8, _D), jnp.float32),
            pltpu.SemaphoreType.DMA,
        ],
    )


@jax.jit
def kernel(layer1, layer0):
    l1f = layer1.reshape(layer1.shape[0], _D)
    l0f = layer0.reshape(layer0.shape[0] * 8, _D)
    out = _make_sc_gather()(l1f, l0f)
    return out.reshape(32, 8, _D)
